# baseline (speedup 1.0000x reference)
import functools

import jax
import jax.numpy as jnp
from jax import lax
from jax.experimental import pallas as pl
from jax.experimental.pallas import tpu as pltpu
from jax.experimental.pallas import tpu_sc as plsc

N_NODES = 10000
N_PAD = 10240
E_EDGES = 320000
NW = 32
EPW = 10240
E_PAD = NW * EPW
CHUNK = 128
CPW = EPW // CHUNK
D = 16
NROWS = N_PAD // 16

_mesh = plsc.VectorSubcoreMesh(core_axis_name="c", subcore_axis_name="s")


def _rsqrt_newton(d):
    seed = jnp.int32(0x5F3759DF) - lax.shift_right_logical(
        lax.bitcast_convert_type(d, jnp.int32), 1
    )
    y = lax.bitcast_convert_type(seed, jnp.float32)
    half_d = d * 0.5
    for _ in range(3):
        y = y * (1.5 - half_d * y * y)
    return jnp.where(d > 0.0, y, 0.0)


def _edge_pipeline(y_sp, dis_sp, agg_sp, rowbuf, colbuf, wbuf, msgbuf, dbuf,
                   gsem, ssem, col_base):
    def fire(k):
        pltpu.async_copy(y_sp.at[rowbuf.at[k]], msgbuf.at[lax.rem(k, 4)], gsem)
        pltpu.async_copy(dis_sp.at[rowbuf.at[k]], dbuf.at[lax.rem(k, 4)], gsem)

    fire(0)
    fire(1)

    def body(k, carry):
        slot = lax.rem(k, 4)

        @pl.when(k >= 2)
        def _():
            km2 = k - 2
            pltpu.make_async_copy(msgbuf.at[lax.rem(km2, 4)],
                                  agg_sp.at[colbuf.at[col_base + km2]],
                                  ssem).wait()

        @pl.when(k + 2 < CPW)
        def _():
            fire(k + 2)

        pltpu.make_async_copy(y_sp.at[rowbuf.at[k]], msgbuf.at[slot],
                              gsem).wait()
        pltpu.make_async_copy(dis_sp.at[rowbuf.at[k]], dbuf.at[slot],
                              gsem).wait()
        for g in range(CHUNK // 16):
            nv = (wbuf[col_base + k, pl.ds(g * 16, 16)]
                  * dbuf[slot, pl.ds(g * 16, 16)])
            for j in range(16):
                e = g * 16 + j
                msgbuf[slot, e, :] = msgbuf[slot, e, :] * nv[j]
        pltpu.async_copy(msgbuf.at[slot], agg_sp.at[colbuf.at[col_base + k]],
                         ssem, add=True)
        return carry

    lax.fori_loop(0, CPW, body, 0)

    def drain(k, carry):
        pltpu.make_async_copy(msgbuf.at[lax.rem(k, 4)],
                              agg_sp.at[colbuf.at[col_base + k]], ssem).wait()
        return carry

    lax.fori_loop(CPW - 2, CPW, drain, 0)


@functools.partial(
    pl.kernel,
    mesh=_mesh,
    out_type=[
        jax.ShapeDtypeStruct((2, N_PAD, D), jnp.float32),
        jax.ShapeDtypeStruct((N_PAD,), jnp.float32),
    ],
    scratch_types=[
        pltpu.VMEM((CPW, CHUNK), jnp.int32),
        pltpu.VMEM((2 * CPW, CHUNK), jnp.int32),
        pltpu.VMEM((2 * CPW, CHUNK), jnp.float32),
        pltpu.VMEM((4, CHUNK, D), jnp.float32),
        pltpu.VMEM((4, CHUNK), jnp.float32),
        pltpu.VMEM((NROWS,), jnp.float32),
        pltpu.VMEM_SHARED((N_PAD, D), jnp.float32),
        pltpu.VMEM_SHARED((N_PAD, D), jnp.float32),
        pltpu.VMEM_SHARED((N_PAD,), jnp.float32),
        pltpu.SemaphoreType.DMA,
        pltpu.SemaphoreType.DMA,
        pltpu.SemaphoreType.DMA,
    ],
    compiler_params=pltpu.CompilerParams(use_tc_tiling_on_sc=False),
)
def _sc_layer1(y_hbm, row_hbm, col_hbm, w_hbm, zero2_hbm, zero1_hbm,
               out_hbm, dis_hbm,
               rowbuf, colbuf, wbuf, msgbuf, dbuf, nodebuf,
               agg_sp, y_sp, deg_sp, stage_sem, gsem, ssem):
    c = lax.axis_index("c")
    s = lax.axis_index("s")
    wid = s * 2 + c

    @pl.when(s == 0)
    def _():
        pltpu.sync_copy(zero2_hbm, agg_sp)

    @pl.when(s == 1)
    def _():
        pltpu.sync_copy(zero1_hbm, deg_sp)

    node_slice = pl.ds(s * NROWS, NROWS)
    pltpu.sync_copy(y_hbm.at[node_slice], y_sp.at[node_slice])

    deg_slice = pl.ds(s * 2 * CPW, 2 * CPW)
    pltpu.async_copy(col_hbm.at[deg_slice], colbuf, stage_sem)
    pltpu.async_copy(w_hbm.at[deg_slice], wbuf, stage_sem)
    row_slice = pl.ds(wid * CPW, CPW)
    pltpu.async_copy(row_hbm.at[row_slice], rowbuf, stage_sem)
    pltpu.make_async_copy(col_hbm.at[deg_slice], colbuf, stage_sem).wait()
    pltpu.make_async_copy(w_hbm.at[deg_slice], wbuf, stage_sem).wait()
    pltpu.make_async_copy(row_hbm.at[row_slice], rowbuf, stage_sem).wait()

    plsc.subcore_barrier()

    def deg_body(k, carry):
        pltpu.async_copy(wbuf.at[k], deg_sp.at[colbuf.at[k]], ssem, add=True)
        return carry

    lax.fori_loop(0, 2 * CPW, deg_body, 0)

    def deg_drain(k, carry):
        pltpu.make_async_copy(wbuf.at[k], deg_sp.at[colbuf.at[k]],
                              ssem).wait()
        return carry

    lax.fori_loop(0, 2 * CPW, deg_drain, 0)

    plsc.subcore_barrier()

    pltpu.sync_copy(deg_sp.at[node_slice], nodebuf)
    for i in range(NROWS // 16):
        nodebuf[pl.ds(i * 16, 16)] = _rsqrt_newton(nodebuf[pl.ds(i * 16, 16)])
    pltpu.sync_copy(nodebuf, deg_sp.at[node_slice])

    @pl.when(c == 0)
    def _():
        pltpu.sync_copy(nodebuf, dis_hbm.at[node_slice])

    plsc.subcore_barrier()

    _edge_pipeline(y_sp, deg_sp, agg_sp, rowbuf, colbuf, wbuf, msgbuf, dbuf,
                   gsem, ssem, CPW * c)

    plsc.subcore_barrier()
    pltpu.sync_copy(
        agg_sp.at[node_slice],
        out_hbm.at[c, node_slice],
    )


@functools.partial(
    pl.kernel,
    mesh=_mesh,
    out_type=jax.ShapeDtypeStruct((2, N_PAD, D), jnp.float32),
    scratch_types=[
        pltpu.VMEM((CPW, CHUNK), jnp.int32),
        pltpu.VMEM((CPW, CHUNK), jnp.int32),
        pltpu.VMEM((CPW, CHUNK), jnp.float32),
        pltpu.VMEM((4, CHUNK, D), jnp.float32),
        pltpu.VMEM((4, CHUNK), jnp.float32),
        pltpu.VMEM_SHARED((N_PAD, D), jnp.float32),
        pltpu.VMEM_SHARED((N_PAD, D), jnp.float32),
        pltpu.VMEM_SHARED((N_PAD,), jnp.float32),
        pltpu.SemaphoreType.DMA,
        pltpu.SemaphoreType.DMA,
        pltpu.SemaphoreType.DMA,
    ],
    compiler_params=pltpu.CompilerParams(use_tc_tiling_on_sc=False),
)
def _sc_layer2(y_hbm, row_hbm, col_hbm, w_hbm, dis_in_hbm, zero2_hbm, out_hbm,
               rowbuf, colbuf, wbuf, msgbuf, dbuf,
               agg_sp, y_sp, dis_sp, stage_sem, gsem, ssem):
    c = lax.axis_index("c")
    s = lax.axis_index("s")
    wid = s * 2 + c

    @pl.when(s == 0)
    def _():
        pltpu.sync_copy(zero2_hbm, agg_sp)

    node_slice = pl.ds(s * NROWS, NROWS)
    pltpu.sync_copy(y_hbm.at[node_slice], y_sp.at[node_slice])
    pltpu.sync_copy(dis_in_hbm.at[node_slice], dis_sp.at[node_slice])

    rows_slice = pl.ds(wid * CPW, CPW)
    pltpu.async_copy(row_hbm.at[rows_slice], rowbuf, stage_sem)
    pltpu.async_copy(col_hbm.at[rows_slice], colbuf, stage_sem)
    pltpu.async_copy(w_hbm.at[rows_slice], wbuf, stage_sem)
    pltpu.make_async_copy(row_hbm.at[rows_slice], rowbuf, stage_sem).wait()
    pltpu.make_async_copy(col_hbm.at[rows_slice], colbuf, stage_sem).wait()
    pltpu.make_async_copy(w_hbm.at[rows_slice], wbuf, stage_sem).wait()

    plsc.subcore_barrier()

    _edge_pipeline(y_sp, dis_sp, agg_sp, rowbuf, colbuf, wbuf, msgbuf, dbuf,
                   gsem, ssem, 0)

    plsc.subcore_barrier()
    pltpu.sync_copy(
        agg_sp.at[node_slice],
        out_hbm.at[c, node_slice],
    )


def _tc1_body(x_ref, w1_ref, y_ref):
    y_ref[...] = jnp.dot(x_ref[...], w1_ref[...],
                         preferred_element_type=jnp.float32)


_tc1 = pl.pallas_call(
    _tc1_body,
    out_shape=jax.ShapeDtypeStruct((N_PAD, D), jnp.float32),
)


def _tc2_body(p_ref, dis_ref, b1_ref, w2_ref, y2_ref):
    dis = dis_ref[...]
    h = jnp.maximum((p_ref[0] + p_ref[1]) * dis + b1_ref[...], 0.0)
    y2_ref[...] = jnp.dot(h, w2_ref[...], preferred_element_type=jnp.float32)


_tc2 = pl.pallas_call(
    _tc2_body,
    out_shape=jax.ShapeDtypeStruct((N_PAD, D), jnp.float32),
)


def _tc3_body(q_ref, dis_ref, b2_ref, out_ref):
    out_ref[...] = jnp.maximum(
        (q_ref[0] + q_ref[1]) * dis_ref[...] + b2_ref[...], 0.0
    )


_tc3 = pl.pallas_call(
    _tc3_body,
    out_shape=jax.ShapeDtypeStruct((N_PAD, D), jnp.float32),
)


@jax.jit
def kernel(x, edge_index, edge_attr, W1, b1, W2, b2):
    row = edge_index[0]
    col = edge_index[1]
    pad_e = E_PAD - E_EDGES
    row_p = jnp.concatenate([row, jnp.zeros((pad_e,), jnp.int32)])
    col_p = jnp.concatenate([col, jnp.zeros((pad_e,), jnp.int32)])
    w_p = jnp.concatenate([edge_attr, jnp.zeros((pad_e,), jnp.float32)])
    row_p = row_p.reshape(NW * CPW, CHUNK)
    col_p = col_p.reshape(NW * CPW, CHUNK)
    w_p = w_p.reshape(NW * CPW, CHUNK)

    x_p = jnp.concatenate(
        [x, jnp.zeros((N_PAD - N_NODES, x.shape[1]), jnp.float32)]
    )
    zero1 = jnp.zeros((N_PAD,), jnp.float32)
    zero2 = jnp.zeros((N_PAD, D), jnp.float32)

    xw1 = _tc1(x_p, W1)
    p1, dis = _sc_layer1(xw1, row_p, col_p, w_p, zero2, zero1)
    dis2 = dis.reshape(N_PAD, 1)
    xw2 = _tc2(p1, dis2, b1.reshape(1, D), W2)
    p2 = _sc_layer2(xw2, row_p, col_p, w_p, dis, zero2)
    out = _tc3(p2, dis2, b2.reshape(1, D))
    return out[:N_NODES]


# speedup vs baseline: 1.0005x; 1.0005x over previous
import functools

import jax
import jax.numpy as jnp
from jax import lax
from jax.experimental import pallas as pl
from jax.experimental.pallas import tpu as pltpu
from jax.experimental.pallas import tpu_sc as plsc

N_NODES = 10000
N_PAD = 10240
E_EDGES = 320000
NW = 32
EPW = 10240
E_PAD = NW * EPW
CHUNK = 128
CPW = EPW // CHUNK
D = 16
NROWS = N_PAD // 16

_mesh = plsc.VectorSubcoreMesh(core_axis_name="c", subcore_axis_name="s")


def _rsqrt_newton(d):
    seed = jnp.int32(0x5F3759DF) - lax.shift_right_logical(
        lax.bitcast_convert_type(d, jnp.int32), 1
    )
    y = lax.bitcast_convert_type(seed, jnp.float32)
    half_d = d * 0.5
    for _ in range(3):
        y = y * (1.5 - half_d * y * y)
    return jnp.where(d > 0.0, y, 0.0)


def _edge_pipeline(y_sp, dis_sp, agg_sp, rowbuf, colbuf, wbuf, msgbuf, dbuf,
                   gsem, ssem, col_base):
    def fire(k):
        pltpu.async_copy(y_sp.at[rowbuf.at[k]], msgbuf.at[lax.rem(k, 4)], gsem)
        pltpu.async_copy(dis_sp.at[rowbuf.at[k]], dbuf.at[lax.rem(k, 4)], gsem)

    fire(0)
    fire(1)

    def body(k, carry):
        slot = lax.rem(k, 4)

        @pl.when(k >= 2)
        def _():
            km2 = k - 2
            pltpu.make_async_copy(msgbuf.at[lax.rem(km2, 4)],
                                  agg_sp.at[colbuf.at[col_base + km2]],
                                  ssem).wait()

        @pl.when(k + 2 < CPW)
        def _():
            fire(k + 2)

        pltpu.make_async_copy(y_sp.at[rowbuf.at[k]], msgbuf.at[slot],
                              gsem).wait()
        pltpu.make_async_copy(dis_sp.at[rowbuf.at[k]], dbuf.at[slot],
                              gsem).wait()
        for g in range(CHUNK // 16):
            nv = (wbuf[col_base + k, pl.ds(g * 16, 16)]
                  * dbuf[slot, pl.ds(g * 16, 16)])
            for j in range(16):
                e = g * 16 + j
                msgbuf[slot, e, :] = msgbuf[slot, e, :] * nv[j]
        pltpu.async_copy(msgbuf.at[slot], agg_sp.at[colbuf.at[col_base + k]],
                         ssem, add=True)
        return carry

    lax.fori_loop(0, CPW, body, 0)

    def drain(k, carry):
        pltpu.make_async_copy(msgbuf.at[lax.rem(k, 4)],
                              agg_sp.at[colbuf.at[col_base + k]], ssem).wait()
        return carry

    lax.fori_loop(CPW - 2, CPW, drain, 0)


@functools.partial(
    pl.kernel,
    mesh=_mesh,
    out_type=[
        jax.ShapeDtypeStruct((2, N_PAD, D), jnp.float32),
        jax.ShapeDtypeStruct((N_PAD,), jnp.float32),
    ],
    scratch_types=[
        pltpu.VMEM((CPW, CHUNK), jnp.int32),
        pltpu.VMEM((2 * CPW, CHUNK), jnp.int32),
        pltpu.VMEM((2 * CPW, CHUNK), jnp.float32),
        pltpu.VMEM((4, CHUNK, D), jnp.float32),
        pltpu.VMEM((4, CHUNK), jnp.float32),
        pltpu.VMEM((NROWS,), jnp.float32),
        pltpu.VMEM_SHARED((N_PAD, D), jnp.float32),
        pltpu.VMEM_SHARED((N_PAD, D), jnp.float32),
        pltpu.VMEM_SHARED((N_PAD,), jnp.float32),
        pltpu.SemaphoreType.DMA,
        pltpu.SemaphoreType.DMA,
        pltpu.SemaphoreType.DMA,
    ],
    compiler_params=pltpu.CompilerParams(use_tc_tiling_on_sc=False),
)
def _sc_layer1(y_hbm, row_hbm, col_hbm, w_hbm, zero2_hbm, zero1_hbm,
               out_hbm, dis_hbm,
               rowbuf, colbuf, wbuf, msgbuf, dbuf, nodebuf,
               agg_sp, y_sp, deg_sp, stage_sem, gsem, ssem):
    c = lax.axis_index("c")
    s = lax.axis_index("s")
    wid = s * 2 + c

    @pl.when(s == 0)
    def _():
        pltpu.sync_copy(zero2_hbm, agg_sp)

    @pl.when(s == 1)
    def _():
        pltpu.sync_copy(zero1_hbm, deg_sp)

    node_slice = pl.ds(s * NROWS, NROWS)
    pltpu.sync_copy(y_hbm.at[node_slice], y_sp.at[node_slice])

    deg_slice = pl.ds(s * 2 * CPW, 2 * CPW)
    pltpu.async_copy(col_hbm.at[deg_slice], colbuf, stage_sem)
    pltpu.async_copy(w_hbm.at[deg_slice], wbuf, stage_sem)
    row_slice = pl.ds(wid * CPW, CPW)
    pltpu.async_copy(row_hbm.at[row_slice], rowbuf, stage_sem)
    pltpu.make_async_copy(col_hbm.at[deg_slice], colbuf, stage_sem).wait()
    pltpu.make_async_copy(w_hbm.at[deg_slice], wbuf, stage_sem).wait()
    pltpu.make_async_copy(row_hbm.at[row_slice], rowbuf, stage_sem).wait()

    plsc.subcore_barrier()

    def deg_body(k, carry):
        pltpu.async_copy(wbuf.at[k], deg_sp.at[colbuf.at[k]], ssem, add=True)
        return carry

    lax.fori_loop(0, 2 * CPW, deg_body, 0)

    pltpu.make_async_copy(col_hbm.at[deg_slice], colbuf, ssem).wait()

    plsc.subcore_barrier()

    pltpu.sync_copy(deg_sp.at[node_slice], nodebuf)
    for i in range(NROWS // 16):
        nodebuf[pl.ds(i * 16, 16)] = _rsqrt_newton(nodebuf[pl.ds(i * 16, 16)])
    pltpu.sync_copy(nodebuf, deg_sp.at[node_slice])

    @pl.when(c == 0)
    def _():
        pltpu.sync_copy(nodebuf, dis_hbm.at[node_slice])

    plsc.subcore_barrier()

    _edge_pipeline(y_sp, deg_sp, agg_sp, rowbuf, colbuf, wbuf, msgbuf, dbuf,
                   gsem, ssem, CPW * c)

    plsc.subcore_barrier()
    pltpu.sync_copy(
        agg_sp.at[node_slice],
        out_hbm.at[c, node_slice],
    )


@functools.partial(
    pl.kernel,
    mesh=_mesh,
    out_type=jax.ShapeDtypeStruct((2, N_PAD, D), jnp.float32),
    scratch_types=[
        pltpu.VMEM((CPW, CHUNK), jnp.int32),
        pltpu.VMEM((CPW, CHUNK), jnp.int32),
        pltpu.VMEM((CPW, CHUNK), jnp.float32),
        pltpu.VMEM((4, CHUNK, D), jnp.float32),
        pltpu.VMEM((4, CHUNK), jnp.float32),
        pltpu.VMEM_SHARED((N_PAD, D), jnp.float32),
        pltpu.VMEM_SHARED((N_PAD, D), jnp.float32),
        pltpu.VMEM_SHARED((N_PAD,), jnp.float32),
        pltpu.SemaphoreType.DMA,
        pltpu.SemaphoreType.DMA,
        pltpu.SemaphoreType.DMA,
    ],
    compiler_params=pltpu.CompilerParams(use_tc_tiling_on_sc=False),
)
def _sc_layer2(y_hbm, row_hbm, col_hbm, w_hbm, dis_in_hbm, zero2_hbm, out_hbm,
               rowbuf, colbuf, wbuf, msgbuf, dbuf,
               agg_sp, y_sp, dis_sp, stage_sem, gsem, ssem):
    c = lax.axis_index("c")
    s = lax.axis_index("s")
    wid = s * 2 + c

    @pl.when(s == 0)
    def _():
        pltpu.sync_copy(zero2_hbm, agg_sp)

    node_slice = pl.ds(s * NROWS, NROWS)
    pltpu.sync_copy(y_hbm.at[node_slice], y_sp.at[node_slice])
    pltpu.sync_copy(dis_in_hbm.at[node_slice], dis_sp.at[node_slice])

    rows_slice = pl.ds(wid * CPW, CPW)
    pltpu.async_copy(row_hbm.at[rows_slice], rowbuf, stage_sem)
    pltpu.async_copy(col_hbm.at[rows_slice], colbuf, stage_sem)
    pltpu.async_copy(w_hbm.at[rows_slice], wbuf, stage_sem)
    pltpu.make_async_copy(row_hbm.at[rows_slice], rowbuf, stage_sem).wait()
    pltpu.make_async_copy(col_hbm.at[rows_slice], colbuf, stage_sem).wait()
    pltpu.make_async_copy(w_hbm.at[rows_slice], wbuf, stage_sem).wait()

    plsc.subcore_barrier()

    _edge_pipeline(y_sp, dis_sp, agg_sp, rowbuf, colbuf, wbuf, msgbuf, dbuf,
                   gsem, ssem, 0)

    plsc.subcore_barrier()
    pltpu.sync_copy(
        agg_sp.at[node_slice],
        out_hbm.at[c, node_slice],
    )


def _tc1_body(x_ref, w1_ref, y_ref):
    y_ref[...] = jnp.dot(x_ref[...], w1_ref[...],
                         preferred_element_type=jnp.float32)


_tc1 = pl.pallas_call(
    _tc1_body,
    out_shape=jax.ShapeDtypeStruct((N_PAD, D), jnp.float32),
)


def _tc2_body(p_ref, dis_ref, b1_ref, w2_ref, y2_ref):
    dis = dis_ref[...]
    h = jnp.maximum((p_ref[0] + p_ref[1]) * dis + b1_ref[...], 0.0)
    y2_ref[...] = jnp.dot(h, w2_ref[...], preferred_element_type=jnp.float32)


_tc2 = pl.pallas_call(
    _tc2_body,
    out_shape=jax.ShapeDtypeStruct((N_PAD, D), jnp.float32),
)


def _tc3_body(q_ref, dis_ref, b2_ref, out_ref):
    out_ref[...] = jnp.maximum(
        (q_ref[0] + q_ref[1]) * dis_ref[...] + b2_ref[...], 0.0
    )


_tc3 = pl.pallas_call(
    _tc3_body,
    out_shape=jax.ShapeDtypeStruct((N_PAD, D), jnp.float32),
)


@jax.jit
def kernel(x, edge_index, edge_attr, W1, b1, W2, b2):
    row = edge_index[0]
    col = edge_index[1]
    pad_e = E_PAD - E_EDGES
    row_p = jnp.concatenate([row, jnp.zeros((pad_e,), jnp.int32)])
    col_p = jnp.concatenate([col, jnp.zeros((pad_e,), jnp.int32)])
    w_p = jnp.concatenate([edge_attr, jnp.zeros((pad_e,), jnp.float32)])
    row_p = row_p.reshape(NW * CPW, CHUNK)
    col_p = col_p.reshape(NW * CPW, CHUNK)
    w_p = w_p.reshape(NW * CPW, CHUNK)

    x_p = jnp.concatenate(
        [x, jnp.zeros((N_PAD - N_NODES, x.shape[1]), jnp.float32)]
    )
    zero1 = jnp.zeros((N_PAD,), jnp.float32)
    zero2 = jnp.zeros((N_PAD, D), jnp.float32)

    xw1 = _tc1(x_p, W1)
    p1, dis = _sc_layer1(xw1, row_p, col_p, w_p, zero2, zero1)
    dis2 = dis.reshape(N_PAD, 1)
    xw2 = _tc2(p1, dis2, b1.reshape(1, D), W2)
    p2 = _sc_layer2(xw2, row_p, col_p, w_p, dis, zero2)
    out = _tc3(p2, dis2, b2.reshape(1, D))
    return out[:N_NODES]


# speedup vs baseline: 1.1654x; 1.1649x over previous
import functools

import jax
import jax.numpy as jnp
from jax import lax
from jax.experimental import pallas as pl
from jax.experimental.pallas import tpu as pltpu
from jax.experimental.pallas import tpu_sc as plsc

N_NODES = 10000
N_PAD = 10240
E_EDGES = 320000
NW = 32
EPW = E_EDGES // NW
CHUNK = 80
CPW = EPW // CHUNK
D = 16
NROWS = N_NODES // 16
DROWS = N_PAD // 16

_mesh = plsc.VectorSubcoreMesh(core_axis_name="c", subcore_axis_name="s")


@functools.partial(
    pl.kernel,
    mesh=_mesh,
    out_type=jax.ShapeDtypeStruct((2, N_PAD), jnp.float32),
    scratch_types=[
        pltpu.VMEM((CPW, CHUNK), jnp.int32),
        pltpu.VMEM((CPW, CHUNK), jnp.float32),
        pltpu.VMEM_SHARED((N_PAD,), jnp.float32),
        pltpu.SemaphoreType.DMA,
        pltpu.SemaphoreType.DMA,
    ],
    compiler_params=pltpu.CompilerParams(use_tc_tiling_on_sc=False),
)
def _sc_degree(col_hbm, w_hbm, zero_hbm, out_hbm, colbuf, wbuf, deg_sp,
               stage_sem, ssem):
    c = lax.axis_index("c")
    s = lax.axis_index("s")
    wid = s * 2 + c

    @pl.when(s == 0)
    def _():
        pltpu.sync_copy(zero_hbm, deg_sp)

    rows_slice = pl.ds(wid * CPW, CPW)
    pltpu.async_copy(col_hbm.at[rows_slice], colbuf, stage_sem)
    pltpu.async_copy(w_hbm.at[rows_slice], wbuf, stage_sem)
    pltpu.make_async_copy(col_hbm.at[rows_slice], colbuf, stage_sem).wait()
    pltpu.make_async_copy(w_hbm.at[rows_slice], wbuf, stage_sem).wait()

    plsc.subcore_barrier()

    def body(k, carry):
        pltpu.async_copy(wbuf.at[k], deg_sp.at[colbuf.at[k]], ssem, add=True)
        return carry

    lax.fori_loop(0, CPW, body, 0)

    pltpu.make_async_copy(w_hbm.at[rows_slice], wbuf, ssem).wait()

    plsc.subcore_barrier()
    pltpu.sync_copy(
        deg_sp.at[pl.ds(s * DROWS, DROWS)],
        out_hbm.at[c, pl.ds(s * DROWS, DROWS)],
    )


@functools.partial(
    pl.kernel,
    mesh=_mesh,
    out_type=jax.ShapeDtypeStruct((2, N_NODES, D), jnp.float32),
    scratch_types=[
        pltpu.VMEM((CPW, CHUNK), jnp.int32),
        pltpu.VMEM((CPW, CHUNK), jnp.int32),
        pltpu.VMEM((CPW, CHUNK), jnp.float32),
        pltpu.VMEM((4, CHUNK, D), jnp.float32),
        pltpu.VMEM_SHARED((N_NODES, D), jnp.float32),
        pltpu.VMEM_SHARED((N_NODES, D), jnp.float32),
        pltpu.SemaphoreType.DMA,
        pltpu.SemaphoreType.DMA,
        pltpu.SemaphoreType.DMA,
    ],
    compiler_params=pltpu.CompilerParams(use_tc_tiling_on_sc=False),
)
def _sc_edge(y_hbm, row_hbm, col_hbm, w_hbm, zero_hbm, out_hbm,
             rowbuf, colbuf, wbuf, msgbuf, agg_sp, y_sp,
             stage_sem, gsem, ssem):
    c = lax.axis_index("c")
    s = lax.axis_index("s")
    wid = s * 2 + c

    @pl.when(s == 0)
    def _():
        pltpu.sync_copy(zero_hbm, agg_sp)

    node_slice = pl.ds(s * NROWS, NROWS)
    pltpu.sync_copy(y_hbm.at[node_slice], y_sp.at[node_slice])

    rows_slice = pl.ds(wid * CPW, CPW)
    pltpu.async_copy(row_hbm.at[rows_slice], rowbuf, stage_sem)
    pltpu.async_copy(col_hbm.at[rows_slice], colbuf, stage_sem)
    pltpu.async_copy(w_hbm.at[rows_slice], wbuf, stage_sem)
    pltpu.make_async_copy(row_hbm.at[rows_slice], rowbuf, stage_sem).wait()
    pltpu.make_async_copy(col_hbm.at[rows_slice], colbuf, stage_sem).wait()
    pltpu.make_async_copy(w_hbm.at[rows_slice], wbuf, stage_sem).wait()

    plsc.subcore_barrier()

    pltpu.async_copy(y_sp.at[rowbuf.at[0]], msgbuf.at[0], gsem)
    pltpu.async_copy(y_sp.at[rowbuf.at[1]], msgbuf.at[1], gsem)

    def body(k, carry):
        slot = lax.rem(k, 4)

        @pl.when(k >= 2)
        def _():
            km2 = k - 2
            pltpu.make_async_copy(msgbuf.at[lax.rem(km2, 4)],
                                  agg_sp.at[colbuf.at[km2]], ssem).wait()

        @pl.when(k + 2 < CPW)
        def _():
            pltpu.async_copy(y_sp.at[rowbuf.at[k + 2]],
                             msgbuf.at[lax.rem(k + 2, 4)], gsem)

        pltpu.make_async_copy(y_sp.at[rowbuf.at[k]], msgbuf.at[slot],
                              gsem).wait()
        for g in range(CHUNK // 16):
            wv = wbuf[k, pl.ds(g * 16, 16)]
            for j in range(16):
                e = g * 16 + j
                msgbuf[slot, e, :] = msgbuf[slot, e, :] * wv[j]
        pltpu.async_copy(msgbuf.at[slot], agg_sp.at[colbuf.at[k]], ssem,
                         add=True)
        return carry

    lax.fori_loop(0, CPW, body, 0)

    def drain(k, carry):
        pltpu.make_async_copy(msgbuf.at[lax.rem(k, 4)],
                              agg_sp.at[colbuf.at[k]], ssem).wait()
        return carry

    lax.fori_loop(CPW - 2, CPW, drain, 0)

    plsc.subcore_barrier()
    pltpu.sync_copy(
        agg_sp.at[node_slice],
        out_hbm.at[c, node_slice],
    )


def _tc1_body(pdeg_ref, x_ref, w1_ref, dis_ref, y_ref):
    deg = pdeg_ref[0] + pdeg_ref[1]
    dis = jnp.where(deg > 0.0, lax.rsqrt(jnp.where(deg > 0.0, deg, 1.0)), 0.0)
    dis_ref[...] = dis
    xw = jnp.dot(x_ref[...], w1_ref[...], preferred_element_type=jnp.float32)
    y_ref[...] = xw * dis[:N_NODES]


_tc1 = pl.pallas_call(
    _tc1_body,
    out_shape=[
        jax.ShapeDtypeStruct((N_PAD, 1), jnp.float32),
        jax.ShapeDtypeStruct((N_NODES, D), jnp.float32),
    ],
)


def _tc2_body(p_ref, dis_ref, b1_ref, w2_ref, y2_ref):
    dis = dis_ref[:N_NODES]
    h = jnp.maximum((p_ref[0] + p_ref[1]) * dis + b1_ref[...], 0.0)
    xw2 = jnp.dot(h, w2_ref[...], preferred_element_type=jnp.float32)
    y2_ref[...] = xw2 * dis


_tc2 = pl.pallas_call(
    _tc2_body,
    out_shape=jax.ShapeDtypeStruct((N_NODES, D), jnp.float32),
)


def _tc3_body(q_ref, dis_ref, b2_ref, out_ref):
    out_ref[...] = jnp.maximum(
        (q_ref[0] + q_ref[1]) * dis_ref[:N_NODES] + b2_ref[...], 0.0
    )


_tc3 = pl.pallas_call(
    _tc3_body,
    out_shape=jax.ShapeDtypeStruct((N_NODES, D), jnp.float32),
)


@jax.jit
def kernel(x, edge_index, edge_attr, W1, b1, W2, b2):
    row_p = edge_index[0].reshape(NW * CPW, CHUNK)
    col_p = edge_index[1].reshape(NW * CPW, CHUNK)
    w_p = edge_attr.reshape(NW * CPW, CHUNK)

    zero1 = jnp.zeros((N_PAD,), jnp.float32)
    zero2 = jnp.zeros((N_NODES, D), jnp.float32)

    pdeg = _sc_degree(col_p, w_p, zero1)
    dis, y1 = _tc1(pdeg.reshape(2, N_PAD, 1), x, W1)

    p1 = _sc_edge(y1, row_p, col_p, w_p, zero2)
    y2 = _tc2(p1, dis, b1.reshape(1, D), W2)

    p2 = _sc_edge(y2, row_p, col_p, w_p, zero2)
    out = _tc3(p2, dis, b2.reshape(1, D))
    return out


# speedup vs baseline: 1.2594x; 1.0806x over previous
import functools

import jax
import jax.numpy as jnp
from jax import lax
from jax.experimental import pallas as pl
from jax.experimental.pallas import tpu as pltpu
from jax.experimental.pallas import tpu_sc as plsc

N_NODES = 10000
N_PAD = 10240
E_EDGES = 320000
NW = 32
EPW = E_EDGES // NW
CHUNK = 80
CPW = EPW // CHUNK
D = 16
NROWS = N_PAD // 16

_mesh = plsc.VectorSubcoreMesh(core_axis_name="c", subcore_axis_name="s")


def _rsqrt_newton(d):
    seed = jnp.int32(0x5F3759DF) - lax.shift_right_logical(
        lax.bitcast_convert_type(d, jnp.int32), 1
    )
    y = lax.bitcast_convert_type(seed, jnp.float32)
    half_d = d * 0.5
    for _ in range(3):
        y = y * (1.5 - half_d * y * y)
    return jnp.where(d > 0.0, y, 0.0)


def _edge_pipeline(y_sp, agg_sp, rowbuf, colbuf, wbuf, msgbuf,
                   gsem, ssem, col_base):
    pltpu.async_copy(y_sp.at[rowbuf.at[0]], msgbuf.at[0], gsem)
    pltpu.async_copy(y_sp.at[rowbuf.at[1]], msgbuf.at[1], gsem)

    def body(k, carry):
        slot = lax.rem(k, 4)

        @pl.when(k >= 2)
        def _():
            km2 = k - 2
            pltpu.make_async_copy(msgbuf.at[lax.rem(km2, 4)],
                                  agg_sp.at[colbuf.at[col_base + km2]],
                                  ssem).wait()

        @pl.when(k + 2 < CPW)
        def _():
            pltpu.async_copy(y_sp.at[rowbuf.at[k + 2]],
                             msgbuf.at[lax.rem(k + 2, 4)], gsem)

        pltpu.make_async_copy(y_sp.at[rowbuf.at[k]], msgbuf.at[slot],
                              gsem).wait()
        for g in range(CHUNK // 16):
            wv = wbuf[col_base + k, pl.ds(g * 16, 16)]
            for j in range(16):
                e = g * 16 + j
                msgbuf[slot, e, :] = msgbuf[slot, e, :] * wv[j]
        pltpu.async_copy(msgbuf.at[slot], agg_sp.at[colbuf.at[col_base + k]],
                         ssem, add=True)
        return carry

    lax.fori_loop(0, CPW, body, 0)

    def drain(k, carry):
        pltpu.make_async_copy(msgbuf.at[lax.rem(k, 4)],
                              agg_sp.at[colbuf.at[col_base + k]], ssem).wait()
        return carry

    lax.fori_loop(CPW - 2, CPW, drain, 0)


@functools.partial(
    pl.kernel,
    mesh=_mesh,
    out_type=[
        jax.ShapeDtypeStruct((2, N_PAD, D), jnp.float32),
        jax.ShapeDtypeStruct((N_PAD,), jnp.float32),
    ],
    scratch_types=[
        pltpu.VMEM((CPW, CHUNK), jnp.int32),
        pltpu.VMEM((2 * CPW, CHUNK), jnp.int32),
        pltpu.VMEM((2 * CPW, CHUNK), jnp.float32),
        pltpu.VMEM((4, CHUNK, D), jnp.float32),
        pltpu.VMEM((NROWS,), jnp.float32),
        pltpu.VMEM((NROWS, D), jnp.float32),
        pltpu.VMEM_SHARED((N_PAD, D), jnp.float32),
        pltpu.VMEM_SHARED((N_PAD, D), jnp.float32),
        pltpu.VMEM_SHARED((N_PAD,), jnp.float32),
        pltpu.SemaphoreType.DMA,
        pltpu.SemaphoreType.DMA,
        pltpu.SemaphoreType.DMA,
    ],
    compiler_params=pltpu.CompilerParams(use_tc_tiling_on_sc=False),
)
def _sc_layer1(xw_hbm, row_hbm, col_hbm, w_hbm, zero2_hbm, zero1_hbm,
               out_hbm, dis_hbm,
               rowbuf, colbuf, wbuf, msgbuf, nodebuf, ybuf,
               agg_sp, y_sp, deg_sp, stage_sem, gsem, ssem):
    c = lax.axis_index("c")
    s = lax.axis_index("s")
    wid = s * 2 + c

    @pl.when(s == 0)
    def _():
        pltpu.sync_copy(zero2_hbm, agg_sp)

    @pl.when(s == 1)
    def _():
        pltpu.sync_copy(zero1_hbm, deg_sp)

    deg_slice = pl.ds(s * 2 * CPW, 2 * CPW)
    pltpu.async_copy(col_hbm.at[deg_slice], colbuf, stage_sem)
    pltpu.async_copy(w_hbm.at[deg_slice], wbuf, stage_sem)
    row_slice = pl.ds(wid * CPW, CPW)
    pltpu.async_copy(row_hbm.at[row_slice], rowbuf, stage_sem)
    pltpu.make_async_copy(col_hbm.at[deg_slice], colbuf, stage_sem).wait()
    pltpu.make_async_copy(w_hbm.at[deg_slice], wbuf, stage_sem).wait()
    pltpu.make_async_copy(row_hbm.at[row_slice], rowbuf, stage_sem).wait()

    plsc.subcore_barrier()

    def deg_body(k, carry):
        pltpu.async_copy(wbuf.at[k], deg_sp.at[colbuf.at[k]], ssem, add=True)
        return carry

    lax.fori_loop(0, 2 * CPW, deg_body, 0)

    pltpu.make_async_copy(w_hbm.at[deg_slice], wbuf, ssem).wait()

    plsc.subcore_barrier()

    node_slice = pl.ds(s * NROWS, NROWS)
    pltpu.sync_copy(deg_sp.at[node_slice], nodebuf)
    for i in range(NROWS // 16):
        nodebuf[pl.ds(i * 16, 16)] = _rsqrt_newton(nodebuf[pl.ds(i * 16, 16)])

    @pl.when(c == 0)
    def _():
        pltpu.sync_copy(nodebuf, dis_hbm.at[node_slice])

    pltpu.sync_copy(xw_hbm.at[node_slice], ybuf)
    for g in range(NROWS // 16):
        dv = nodebuf[pl.ds(g * 16, 16)]
        for j in range(16):
            n = g * 16 + j
            ybuf[n, :] = ybuf[n, :] * dv[j]
    pltpu.sync_copy(ybuf, y_sp.at[node_slice])

    plsc.subcore_barrier()

    _edge_pipeline(y_sp, agg_sp, rowbuf, colbuf, wbuf, msgbuf,
                   gsem, ssem, CPW * c)

    plsc.subcore_barrier()
    pltpu.sync_copy(
        agg_sp.at[node_slice],
        out_hbm.at[c, node_slice],
    )


@functools.partial(
    pl.kernel,
    mesh=_mesh,
    out_type=jax.ShapeDtypeStruct((2, N_PAD, D), jnp.float32),
    scratch_types=[
        pltpu.VMEM((CPW, CHUNK), jnp.int32),
        pltpu.VMEM((CPW, CHUNK), jnp.int32),
        pltpu.VMEM((CPW, CHUNK), jnp.float32),
        pltpu.VMEM((4, CHUNK, D), jnp.float32),
        pltpu.VMEM_SHARED((N_PAD, D), jnp.float32),
        pltpu.VMEM_SHARED((N_PAD, D), jnp.float32),
        pltpu.SemaphoreType.DMA,
        pltpu.SemaphoreType.DMA,
        pltpu.SemaphoreType.DMA,
    ],
    compiler_params=pltpu.CompilerParams(use_tc_tiling_on_sc=False),
)
def _sc_edge(y_hbm, row_hbm, col_hbm, w_hbm, zero_hbm, out_hbm,
             rowbuf, colbuf, wbuf, msgbuf, agg_sp, y_sp,
             stage_sem, gsem, ssem):
    c = lax.axis_index("c")
    s = lax.axis_index("s")
    wid = s * 2 + c

    @pl.when(s == 0)
    def _():
        pltpu.sync_copy(zero_hbm, agg_sp)

    node_slice = pl.ds(s * NROWS, NROWS)
    pltpu.sync_copy(y_hbm.at[node_slice], y_sp.at[node_slice])

    rows_slice = pl.ds(wid * CPW, CPW)
    pltpu.async_copy(row_hbm.at[rows_slice], rowbuf, stage_sem)
    pltpu.async_copy(col_hbm.at[rows_slice], colbuf, stage_sem)
    pltpu.async_copy(w_hbm.at[rows_slice], wbuf, stage_sem)
    pltpu.make_async_copy(row_hbm.at[rows_slice], rowbuf, stage_sem).wait()
    pltpu.make_async_copy(col_hbm.at[rows_slice], colbuf, stage_sem).wait()
    pltpu.make_async_copy(w_hbm.at[rows_slice], wbuf, stage_sem).wait()

    plsc.subcore_barrier()

    _edge_pipeline(y_sp, agg_sp, rowbuf, colbuf, wbuf, msgbuf,
                   gsem, ssem, 0)

    plsc.subcore_barrier()
    pltpu.sync_copy(
        agg_sp.at[node_slice],
        out_hbm.at[c, node_slice],
    )


def _tc1_body(x_ref, w1_ref, y_ref):
    xw = jnp.dot(x_ref[...], w1_ref[...], preferred_element_type=jnp.float32)
    y_ref[...] = jnp.concatenate(
        [xw, jnp.zeros((N_PAD - N_NODES, D), jnp.float32)], axis=0
    )


_tc1 = pl.pallas_call(
    _tc1_body,
    out_shape=jax.ShapeDtypeStruct((N_PAD, D), jnp.float32),
)


def _tc2_body(p_ref, dis_ref, b1_ref, w2_ref, y2_ref):
    dis = dis_ref[...]
    h = jnp.maximum((p_ref[0] + p_ref[1]) * dis + b1_ref[...], 0.0)
    xw2 = jnp.dot(h, w2_ref[...], preferred_element_type=jnp.float32)
    y2_ref[...] = xw2 * dis


_tc2 = pl.pallas_call(
    _tc2_body,
    out_shape=jax.ShapeDtypeStruct((N_PAD, D), jnp.float32),
)


def _tc3_body(q_ref, dis_ref, b2_ref, out_ref):
    full = jnp.maximum(
        (q_ref[0] + q_ref[1]) * dis_ref[...] + b2_ref[...], 0.0
    )
    out_ref[...] = full[:N_NODES]


_tc3 = pl.pallas_call(
    _tc3_body,
    out_shape=jax.ShapeDtypeStruct((N_NODES, D), jnp.float32),
)


@jax.jit
def kernel(x, edge_index, edge_attr, W1, b1, W2, b2):
    row_p = edge_index[0].reshape(NW * CPW, CHUNK)
    col_p = edge_index[1].reshape(NW * CPW, CHUNK)
    w_p = edge_attr.reshape(NW * CPW, CHUNK)

    zero1 = jnp.zeros((N_PAD,), jnp.float32)
    zero2 = jnp.zeros((N_PAD, D), jnp.float32)

    xw1 = _tc1(x, W1)
    p1, dis = _sc_layer1(xw1, row_p, col_p, w_p, zero2, zero1)
    dis2 = dis.reshape(N_PAD, 1)
    y2 = _tc2(p1, dis2, b1.reshape(1, D), W2)
    p2 = _sc_edge(y2, row_p, col_p, w_p, zero2)
    out = _tc3(p2, dis2, b2.reshape(1, D))
    return out
